# Initial kernel scaffold; baseline (speedup 1.0000x reference)
import jax
import jax.numpy as jnp
from jax.experimental import pallas as pl


def kernel(inputs, table):
    raise NotImplementedError("write your pallas kernel here")


# speedup vs baseline: 4.9499x; 4.9499x over previous
import functools

import jax
import jax.numpy as jnp
from jax import lax
from jax.experimental import pallas as pl
from jax.experimental.pallas import tpu as pltpu
from jax.experimental.pallas import tpu_sc as plsc

BATCH = 16384
HIST = 200
EMBED_DIM = 32

_B = BATCH * HIST
_NC, _NS = 2, 16
_NW = _NC * _NS
_IW = 128
_K = 8
_CHUNK = _IW * _K
_B_PER_W = _B // _NW
_N_CHUNKS = _B_PER_W // _CHUNK


def _emb_kernel(idx_hbm, tab_hbm, out_hbm, idx_v, rows_v, sems):
    wid = lax.axis_index("s") * _NC + lax.axis_index("c")
    irow_base = wid * (_B_PER_W // _IW)
    row_base = wid * _B_PER_W

    def fetch(c, buf):
        pltpu.sync_copy(idx_hbm.at[pl.ds(irow_base + c * _K, _K)], idx_v.at[buf])
        for j in range(_K):
            pltpu.async_copy(
                tab_hbm.at[idx_v.at[buf, j]],
                rows_v.at[buf, pl.ds(j * _IW, _IW)],
                sems.at[buf],
            )

    def drain(buf):
        for j in range(_K):
            pltpu.make_async_copy(
                tab_hbm.at[idx_v.at[buf, j]],
                rows_v.at[buf, pl.ds(j * _IW, _IW)],
                sems.at[buf],
            ).wait()

    fetch(0, 0)

    @pl.loop(0, _N_CHUNKS)
    def _chunk(c):
        buf = lax.rem(c, 2)
        nbuf = 1 - buf

        @pl.when(c + 1 < _N_CHUNKS)
        def _():
            fetch(c + 1, nbuf)

        drain(buf)
        pltpu.sync_copy(
            rows_v.at[buf],
            out_hbm.at[pl.ds(row_base + c * _CHUNK, _CHUNK)],
        )


def kernel(inputs, table):
    idx = inputs.reshape(_B // _IW, _IW)
    mesh = plsc.VectorSubcoreMesh(core_axis_name="c", subcore_axis_name="s")
    run = functools.partial(
        pl.kernel,
        out_type=jax.ShapeDtypeStruct((_B, EMBED_DIM), jnp.float32),
        mesh=mesh,
        scratch_types=[
            pltpu.VMEM((2, _K, _IW), jnp.int32),
            pltpu.VMEM((2, _CHUNK, EMBED_DIM), jnp.float32),
            pltpu.SemaphoreType.DMA((2,)),
        ],
        compiler_params=pltpu.CompilerParams(use_tc_tiling_on_sc=False),
    )(_emb_kernel)
    out = run(idx, table)
    return out.reshape(BATCH, HIST, EMBED_DIM)


# speedup vs baseline: 5.0528x; 1.0208x over previous
import functools

import jax
import jax.numpy as jnp
from jax import lax
from jax.experimental import pallas as pl
from jax.experimental.pallas import tpu as pltpu
from jax.experimental.pallas import tpu_sc as plsc

BATCH = 16384
HIST = 200
EMBED_DIM = 32

_B = BATCH * HIST
_NC, _NS = 2, 16
_NW = _NC * _NS
_IW = 128
_K = 8
_CHUNK = _IW * _K
_B_PER_W = _B // _NW
_N_CHUNKS = _B_PER_W // _CHUNK
_NBUF = 3


def _emb_kernel(idx_hbm, tab_hbm, out_hbm, idx_v, rows_v, idx_sems, g_sems,
                wb_sems):
    wid = lax.axis_index("s") * _NC + lax.axis_index("c")
    irow_base = wid * (_B_PER_W // _IW)
    row_base = wid * _B_PER_W

    def idx_copy(c):
        buf = lax.rem(c, _NBUF)
        return pltpu.make_async_copy(
            idx_hbm.at[pl.ds(irow_base + c * _K, _K)],
            idx_v.at[buf],
            idx_sems.at[buf],
        )

    def gather(c, j):
        buf = lax.rem(c, _NBUF)
        return pltpu.make_async_copy(
            tab_hbm.at[idx_v.at[buf, j]],
            rows_v.at[buf, pl.ds(j * _IW, _IW)],
            g_sems.at[buf],
        )

    def writeback(c):
        buf = lax.rem(c, _NBUF)
        return pltpu.make_async_copy(
            rows_v.at[buf],
            out_hbm.at[pl.ds(row_base + c * _CHUNK, _CHUNK)],
            wb_sems.at[buf],
        )

    def fire_gathers(c):
        idx_copy(c).wait()
        for j in range(_K):
            gather(c, j).start()

    idx_copy(0).start()
    idx_copy(1).start()
    fire_gathers(0)

    @pl.loop(0, _N_CHUNKS)
    def _chunk(c):
        @pl.when(c + 2 < _N_CHUNKS)
        def _():
            idx_copy(c + 2).start()

        @pl.when(c + 1 < _N_CHUNKS)
        def _():
            @pl.when(c >= 2)
            def _():
                writeback(c - 2).wait()

            fire_gathers(c + 1)

        for j in range(_K):
            gather(c, j).wait()
        writeback(c).start()

    writeback(_N_CHUNKS - 3).wait()
    writeback(_N_CHUNKS - 2).wait()
    writeback(_N_CHUNKS - 1).wait()


def kernel(inputs, table):
    idx = inputs.reshape(_B // _IW, _IW)
    mesh = plsc.VectorSubcoreMesh(core_axis_name="c", subcore_axis_name="s")
    run = functools.partial(
        pl.kernel,
        out_type=jax.ShapeDtypeStruct((_B, EMBED_DIM), jnp.float32),
        mesh=mesh,
        scratch_types=[
            pltpu.VMEM((_NBUF, _K, _IW), jnp.int32),
            pltpu.VMEM((_NBUF, _CHUNK, EMBED_DIM), jnp.float32),
            pltpu.SemaphoreType.DMA((_NBUF,)),
            pltpu.SemaphoreType.DMA((_NBUF,)),
            pltpu.SemaphoreType.DMA((_NBUF,)),
        ],
        compiler_params=pltpu.CompilerParams(use_tc_tiling_on_sc=False),
    )(_emb_kernel)
    out = run(idx, table)
    return out.reshape(BATCH, HIST, EMBED_DIM)
